# baseline (speedup 1.0000x reference)
import functools

import jax
import jax.numpy as jnp
from jax import lax
from jax.experimental import pallas as pl
from jax.experimental.pallas import tpu as pltpu
from jax.experimental.pallas import tpu_sc as plsc

KE_DIM = 64
MARGIN_V = 1.0
LANES = 16
NWORK = 32
CHUNK = 128
POS_N = 16384
PER_W = POS_N // NWORK
NCHUNK = PER_W // CHUNK


def _make_sc_call():
    mesh = plsc.VectorSubcoreMesh(core_axis_name="c", subcore_axis_name="s")

    row_t = pltpu.VMEM((CHUNK, KE_DIM), jnp.float32)
    idx_t = pltpu.VMEM((CHUNK,), jnp.int32)

    @functools.partial(
        pl.kernel,
        mesh=mesh,
        compiler_params=pltpu.CompilerParams(use_tc_tiling_on_sc=False),
        out_type=jax.ShapeDtypeStruct((NWORK, LANES), jnp.float32),
        scratch_types=[
            idx_t, idx_t, idx_t,
            idx_t, idx_t, idx_t,
            row_t, row_t, row_t, row_t, row_t, row_t,
            row_t, row_t, row_t, row_t, row_t, row_t,
            pltpu.VMEM((LANES,), jnp.float32),
            pltpu.SemaphoreType.DMA,
        ],
    )
    def sc_fn(eEmb, eCov, rEmb, rCov, hIdx, rIdx, tIdx, out,
              phix, prix, ptix, nhix, nrix, ntix,
              phm, phv, ptm, ptv, prm, prv,
              nhm, nhv, ntm, ntv, nrm, nrv,
              accv, sem):
        cid = lax.axis_index("c")
        sid = lax.axis_index("s")
        wid = sid * 2 + cid
        base = wid * PER_W

        iota = lax.iota(jnp.int32, LANES)
        one = jnp.float32(1.0)

        def triple_score(hm, hv, tm, tv, rm, rv, t):
            s = jnp.zeros((LANES,), jnp.float32)
            for g in range(KE_DIM // LANES):
                sl = pl.ds(g * LANES, LANES)
                hm_v = hm[t, sl]
                hv_v = hv[t, sl]
                tm_v = tm[t, sl]
                tv_v = tv[t, sl]
                rm_v = rm[t, sl]
                rv_v = rv[t, sl]
                ev = tv_v + hv_v
                d = rm_v - (tm_v - hm_v)
                dd = d * d
                s = s + (ev + dd) * (one / rv_v) + (rv_v + dd) * (one / ev)
            p0 = s[0] + s[1]
            p1 = s[2] + s[3]
            p2 = s[4] + s[5]
            p3 = s[6] + s[7]
            p4 = s[8] + s[9]
            p5 = s[10] + s[11]
            p6 = s[12] + s[13]
            p7 = s[14] + s[15]
            tot = ((p0 + p1) + (p2 + p3)) + ((p4 + p5) + (p6 + p7))
            return (tot - jnp.float32(2.0 * KE_DIM)) * jnp.float32(0.25)

        loss = jnp.float32(0.0)
        for c in range(NCHUNK):
            pos_off = base + c * CHUNK
            neg_off = pos_off + POS_N
            pltpu.sync_copy(hIdx.at[pl.ds(pos_off, CHUNK)], phix)
            pltpu.sync_copy(rIdx.at[pl.ds(pos_off, CHUNK)], prix)
            pltpu.sync_copy(tIdx.at[pl.ds(pos_off, CHUNK)], ptix)
            pltpu.sync_copy(hIdx.at[pl.ds(neg_off, CHUNK)], nhix)
            pltpu.sync_copy(rIdx.at[pl.ds(neg_off, CHUNK)], nrix)
            pltpu.sync_copy(tIdx.at[pl.ds(neg_off, CHUNK)], ntix)
            cps = [
                pltpu.async_copy(eEmb.at[phix], phm, sem),
                pltpu.async_copy(eCov.at[phix], phv, sem),
                pltpu.async_copy(eEmb.at[ptix], ptm, sem),
                pltpu.async_copy(eCov.at[ptix], ptv, sem),
                pltpu.async_copy(rEmb.at[prix], prm, sem),
                pltpu.async_copy(rCov.at[prix], prv, sem),
                pltpu.async_copy(eEmb.at[nhix], nhm, sem),
                pltpu.async_copy(eCov.at[nhix], nhv, sem),
                pltpu.async_copy(eEmb.at[ntix], ntm, sem),
                pltpu.async_copy(eCov.at[ntix], ntv, sem),
                pltpu.async_copy(rEmb.at[nrix], nrm, sem),
                pltpu.async_copy(rCov.at[nrix], nrv, sem),
            ]
            for cp in cps:
                cp.wait()

            def body(t, carry):
                sp = triple_score(phm, phv, ptm, ptv, prm, prv, t)
                sn = triple_score(nhm, nhv, ntm, ntv, nrm, nrv, t)
                return carry + jnp.maximum(sp - sn + jnp.float32(MARGIN_V),
                                           jnp.float32(0.0))

            loss = lax.fori_loop(0, CHUNK, body, loss)

        accv[...] = jnp.where(iota == 0, loss, jnp.float32(0.0))
        pltpu.sync_copy(accv, out.at[wid])

    return sc_fn


_SC_FN = _make_sc_call()


@jax.jit
def kernel(posX, negX, entityEmbed, entityCovar, relationEmbed, relationCovar):
    X = jnp.concatenate([posX, negX], axis=0)
    h = X[:, 0]
    r = X[:, 1]
    t = X[:, 2]
    partials = _SC_FN(entityEmbed, entityCovar, relationEmbed, relationCovar,
                      h, r, t)
    return jnp.sum(partials) / jnp.float32(posX.shape[0])
